# Initial kernel scaffold; baseline (speedup 1.0000x reference)
import jax
import jax.numpy as jnp
from jax.experimental import pallas as pl


def kernel(x, proxies, W_gat, att_src, att_dst, b_gat, W1, b1, W2, b2, Wf, bf):
    raise NotImplementedError("write your pallas kernel here")


# speedup vs baseline: 238.7533x; 238.7533x over previous
import jax
import jax.numpy as jnp
from jax.experimental import pallas as pl

P = 8
N = 10000
EMBED = 128
H = 2
C = H * EMBED
GOUT = H * C
HID = 4 * EMBED
OUTD = 128

BLOCK = 1000


def _leaky(v):
    return jnp.where(v >= 0, v, 0.2 * v)


def _fused_kernel(x_ref, p_ref, wg_ref, asrc_ref, adst_ref, bg_ref,
                  w1_ref, b1_ref, w2_ref, b2_ref, wf_ref, bf_ref,
                  preds_ref, f_ref):
    xb = x_ref[...]
    wg = wg_ref[...]
    h = jnp.dot(xb, wg, preferred_element_type=jnp.float32)
    hp = jnp.dot(p_ref[...], wg, preferred_element_type=jnp.float32)

    asrc = asrc_ref[...]
    adst = adst_ref[...]

    ts = h * asrc
    a_src0 = jnp.sum(ts[:, :C], axis=1, keepdims=True)
    a_src1 = jnp.sum(ts[:, C:], axis=1, keepdims=True)
    td = h * adst
    a_dst0 = jnp.sum(td[:, :C], axis=1, keepdims=True)
    a_dst1 = jnp.sum(td[:, C:], axis=1, keepdims=True)

    tp = hp * asrc
    ap0 = jnp.sum(tp[:, :C], axis=1)
    ap1 = jnp.sum(tp[:, C:], axis=1)

    def head_agg(ap, a_src_h, a_dst_h, hp_h, h_h):
        logits = _leaky(ap[None, :] + a_dst_h)
        logit_self = _leaky(a_src_h + a_dst_h)
        m = jnp.maximum(jnp.max(logits, axis=1, keepdims=True), logit_self)
        e = jnp.exp(logits - m)
        es = jnp.exp(logit_self - m)
        den = jnp.sum(e, axis=1, keepdims=True) + es + 1e-16
        w = e / den
        ws = es / den
        return jnp.dot(w, hp_h, preferred_element_type=jnp.float32) + ws * h_h

    agg0 = head_agg(ap0, a_src0, a_dst0, hp[:, :C], h[:, :C])
    agg1 = head_agg(ap1, a_src1, a_dst1, hp[:, C:], h[:, C:])

    gat = jnp.concatenate([agg0, agg1], axis=1) + bg_ref[...]
    f0 = jnp.maximum(gat, 0.0)
    f1 = jnp.maximum(jnp.dot(f0, w1_ref[...], preferred_element_type=jnp.float32) + b1_ref[...], 0.0)
    f2 = jnp.maximum(jnp.dot(f1, w2_ref[...], preferred_element_type=jnp.float32) + b2_ref[...], 0.0)
    f_ref[...] = f2
    preds_ref[...] = jnp.dot(f2, wf_ref[...], preferred_element_type=jnp.float32) + bf_ref[...]


def kernel(x, proxies, W_gat, att_src, att_dst, b_gat, W1, b1, W2, b2, Wf, bf):
    grid = (N // BLOCK,)
    row_spec = pl.BlockSpec((BLOCK, EMBED), lambda i: (i, 0))
    out_row_spec = pl.BlockSpec((BLOCK, OUTD), lambda i: (i, 0))

    def full(shape):
        return pl.BlockSpec(shape, lambda i: tuple(0 for _ in shape))

    preds, f = pl.pallas_call(
        _fused_kernel,
        grid=grid,
        in_specs=[
            row_spec,
            full((P, EMBED)),
            full((EMBED, GOUT)),
            full((1, GOUT)),
            full((1, GOUT)),
            full((1, GOUT)),
            full((GOUT, HID)),
            full((1, HID)),
            full((HID, EMBED)),
            full((1, EMBED)),
            full((EMBED, OUTD)),
            full((1, OUTD)),
        ],
        out_specs=[out_row_spec, pl.BlockSpec((BLOCK, EMBED), lambda i: (i, 0))],
        out_shape=[
            jax.ShapeDtypeStruct((N, OUTD), jnp.float32),
            jax.ShapeDtypeStruct((N, EMBED), jnp.float32),
        ],
    )(
        x,
        proxies,
        W_gat,
        att_src.reshape(1, GOUT),
        att_dst.reshape(1, GOUT),
        b_gat.reshape(1, GOUT),
        W1,
        b1.reshape(1, HID),
        W2,
        b2.reshape(1, EMBED),
        Wf,
        bf.reshape(1, OUTD),
    )
    return preds, f


# speedup vs baseline: 464.7763x; 1.9467x over previous
import numpy as np
import jax
import jax.numpy as jnp
from jax.experimental import pallas as pl

P = 8
N = 10000
EMBED = 128
H = 2
C = H * EMBED
GOUT = H * C
HID = 4 * EMBED
OUTD = 128

BLOCK = 1000
W = 32
NEG = -1e30

_S_np = np.zeros((W, 8), np.float32)
_S_np[0:9, 0] = 1.0
_S_np[16:25, 1] = 1.0
_ST_np = np.zeros((8, W), np.float32)
_ST_np[0, 0:9] = 1.0
_ST_np[1, 16:25] = 1.0
_E_np = np.zeros((W, P), np.float32)
_E_np[0:8, :] = np.eye(P, dtype=np.float32)
_E_np[16:24, :] = np.eye(P, dtype=np.float32)
_HPmask_np = np.zeros((W, GOUT), np.float32)
_HPmask_np[0:8, :C] = 1.0
_HPmask_np[16:24, C:] = 1.0
_Ssel_np = np.zeros((W, GOUT), np.float32)
_Ssel_np[8, :C] = 1.0
_Ssel_np[24, C:] = 1.0
_PAD_np = np.full((1, W), NEG, np.float32)
_PAD_np[0, 0:9] = 0.0
_PAD_np[0, 16:25] = 0.0
_G0_np = np.zeros((P, W), np.float32)
_G0_np[:, 0:8] = np.eye(P, dtype=np.float32)
_G1_np = np.zeros((P, W), np.float32)
_G1_np[:, 16:24] = np.eye(P, dtype=np.float32)


def _fused_kernel(x_ref, p_ref, wg_ref, m2_ref, as0_ref, as1_ref,
                  g0_ref, g1_ref, pad_ref, s_ref, st_ref, e_ref,
                  hpm_ref, ssel_ref, bg_ref,
                  w1_ref, b1_ref, w2_ref, b2_ref, wf_ref, bf_ref,
                  preds_ref, f_ref):
    f32 = jnp.float32
    xb = x_ref[...]
    wg = wg_ref[...]
    h = jnp.dot(xb, wg, preferred_element_type=f32)
    hp = jnp.dot(p_ref[...], wg, preferred_element_type=f32)

    dn = (((1,), (1,)), ((), ()))
    ap0 = jax.lax.dot_general(as0_ref[...], hp, dn, preferred_element_type=f32)
    ap1 = jax.lax.dot_general(as1_ref[...], hp, dn, preferred_element_type=f32)
    apvec = (jnp.dot(ap0, g0_ref[...], preferred_element_type=f32)
             + jnp.dot(ap1, g1_ref[...], preferred_element_type=f32)
             + pad_ref[...])

    z = jnp.dot(h, m2_ref[...], preferred_element_type=f32) + apvec
    logits = jnp.maximum(z, 0.2 * z)
    m = jnp.max(logits, axis=1, keepdims=True)
    e = jnp.exp(logits - m)
    den = jnp.dot(e, s_ref[...], preferred_element_type=f32) + 1e-16
    rc = jnp.dot(1.0 / den, st_ref[...], preferred_element_type=f32)
    w = e * rc

    hpw = jnp.dot(e_ref[...], hp, preferred_element_type=f32) * hpm_ref[...]
    agg = jnp.dot(w, hpw, preferred_element_type=f32)
    wsb = jnp.dot(w, ssel_ref[...], preferred_element_type=f32)
    gat = agg + wsb * h + bg_ref[...]

    f0 = jnp.maximum(gat, 0.0)
    f1 = jnp.maximum(jnp.dot(f0, w1_ref[...], preferred_element_type=f32) + b1_ref[...], 0.0)
    f2 = jnp.maximum(jnp.dot(f1, w2_ref[...], preferred_element_type=f32) + b2_ref[...], 0.0)
    f_ref[...] = f2
    preds_ref[...] = jnp.dot(f2, wf_ref[...], preferred_element_type=f32) + bf_ref[...]


def kernel(x, proxies, W_gat, att_src, att_dst, b_gat, W1, b1, W2, b2, Wf, bf):
    asrc_f = att_src.reshape(GOUT)
    adst_f = att_dst.reshape(GOUT)
    m0 = np.zeros((GOUT,), np.float32)
    m0[:C] = 1.0
    m0 = jnp.asarray(m0)
    m1 = 1.0 - m0

    M2 = jnp.zeros((GOUT, W), jnp.float32)
    M2 = M2.at[:, 0:8].set(jnp.broadcast_to((adst_f * m0)[:, None], (GOUT, 8)))
    M2 = M2.at[:, 8].set((asrc_f + adst_f) * m0)
    M2 = M2.at[:, 16:24].set(jnp.broadcast_to((adst_f * m1)[:, None], (GOUT, 8)))
    M2 = M2.at[:, 24].set((asrc_f + adst_f) * m1)

    as0 = (asrc_f * m0).reshape(1, GOUT)
    as1 = (asrc_f * m1).reshape(1, GOUT)

    grid = (N // BLOCK,)
    row_spec = pl.BlockSpec((BLOCK, EMBED), lambda i: (i, 0))

    def full(shape):
        return pl.BlockSpec(shape, lambda i: tuple(0 for _ in shape))

    preds, f = pl.pallas_call(
        _fused_kernel,
        grid=grid,
        in_specs=[
            row_spec,
            full((P, EMBED)),
            full((EMBED, GOUT)),
            full((GOUT, W)),
            full((1, GOUT)),
            full((1, GOUT)),
            full((P, W)),
            full((P, W)),
            full((1, W)),
            full((W, 8)),
            full((8, W)),
            full((W, P)),
            full((W, GOUT)),
            full((W, GOUT)),
            full((1, GOUT)),
            full((GOUT, HID)),
            full((1, HID)),
            full((HID, EMBED)),
            full((1, EMBED)),
            full((EMBED, OUTD)),
            full((1, OUTD)),
        ],
        out_specs=[pl.BlockSpec((BLOCK, OUTD), lambda i: (i, 0)),
                   pl.BlockSpec((BLOCK, EMBED), lambda i: (i, 0))],
        out_shape=[
            jax.ShapeDtypeStruct((N, OUTD), jnp.float32),
            jax.ShapeDtypeStruct((N, EMBED), jnp.float32),
        ],
    )(
        x,
        proxies,
        W_gat,
        M2,
        as0,
        as1,
        jnp.asarray(_G0_np),
        jnp.asarray(_G1_np),
        jnp.asarray(_PAD_np),
        jnp.asarray(_S_np),
        jnp.asarray(_ST_np),
        jnp.asarray(_E_np),
        jnp.asarray(_HPmask_np),
        jnp.asarray(_Ssel_np),
        b_gat.reshape(1, GOUT),
        W1,
        b1.reshape(1, HID),
        W2,
        b2.reshape(1, EMBED),
        Wf,
        bf.reshape(1, OUTD),
    )
    return preds, f


# speedup vs baseline: 560.3442x; 1.2056x over previous
import numpy as np
import jax
import jax.numpy as jnp
from jax.experimental import pallas as pl
from jax.experimental.pallas import tpu as pltpu

P = 8
N = 10000
EMBED = 128
H = 2
C = H * EMBED
GOUT = H * C
HID = 4 * EMBED
OUTD = 128

BLOCK = 1000
W = 32
NEG = -1e30

_S_np = np.zeros((W, 8), np.float32)
_S_np[0:9, 0] = 1.0
_S_np[16:25, 1] = 1.0
_ST_np = np.zeros((8, W), np.float32)
_ST_np[0, 0:9] = 1.0
_ST_np[1, 16:25] = 1.0
_E_np = np.zeros((W, P), np.float32)
_E_np[0:8, :] = np.eye(P, dtype=np.float32)
_E_np[16:24, :] = np.eye(P, dtype=np.float32)
_HPmask_np = np.zeros((W, GOUT), np.float32)
_HPmask_np[0:8, :C] = 1.0
_HPmask_np[16:24, C:] = 1.0
_Ssel_np = np.zeros((W, GOUT), np.float32)
_Ssel_np[8, :C] = 1.0
_Ssel_np[24, C:] = 1.0
_PAD_np = np.full((1, W), NEG, np.float32)
_PAD_np[0, 0:9] = 0.0
_PAD_np[0, 16:25] = 0.0
_G0_np = np.zeros((P, W), np.float32)
_G0_np[:, 0:8] = np.eye(P, dtype=np.float32)
_G1_np = np.zeros((P, W), np.float32)
_G1_np[:, 16:24] = np.eye(P, dtype=np.float32)


def _fused_kernel(x_ref, p_ref, wg_ref, m2_ref, as0_ref, as1_ref,
                  g0_ref, g1_ref, pad_ref, s_ref, st_ref, e_ref,
                  hpm_ref, ssel_ref, bg_ref,
                  w1_ref, b1_ref, w2_ref, b2_ref, wf_ref, bf_ref,
                  preds_ref, f_ref,
                  wgm2_scr, hpw2_scr, apvec_scr):
    f32 = jnp.float32

    @pl.when(pl.program_id(0) == 0)
    def _prep():
        wg = wg_ref[...]
        hp = jnp.dot(p_ref[...], wg, preferred_element_type=f32)
        dn = (((1,), (1,)), ((), ()))
        ap0 = jax.lax.dot_general(as0_ref[...], hp, dn, preferred_element_type=f32)
        ap1 = jax.lax.dot_general(as1_ref[...], hp, dn, preferred_element_type=f32)
        apvec_scr[...] = (jnp.dot(ap0, g0_ref[...], preferred_element_type=f32)
                          + jnp.dot(ap1, g1_ref[...], preferred_element_type=f32)
                          + pad_ref[...])
        wgm2_scr[...] = jnp.dot(wg, m2_ref[...], preferred_element_type=f32)
        hpw = jnp.dot(e_ref[...], hp, preferred_element_type=f32) * hpm_ref[...]
        hpw2_scr[...] = jnp.concatenate([hpw, ssel_ref[...]], axis=1)

    xb = x_ref[...]
    h = jnp.dot(xb, wg_ref[...], preferred_element_type=f32)

    z = jnp.dot(xb, wgm2_scr[...], preferred_element_type=f32) + apvec_scr[...]
    logits = jnp.maximum(z, 0.2 * z)
    m = jnp.max(logits, axis=1, keepdims=True)
    e = jnp.exp(logits - m)
    den = jnp.dot(e, s_ref[...], preferred_element_type=f32) + 1e-16
    rc = jnp.dot(1.0 / den, st_ref[...], preferred_element_type=f32)
    w = e * rc

    out = jnp.dot(w, hpw2_scr[...], preferred_element_type=f32)
    gat = out[:, :GOUT] + out[:, GOUT:] * h + bg_ref[...]

    f0 = jnp.maximum(gat, 0.0)
    f1 = jnp.maximum(jnp.dot(f0, w1_ref[...], preferred_element_type=f32) + b1_ref[...], 0.0)
    f2 = jnp.maximum(jnp.dot(f1, w2_ref[...], preferred_element_type=f32) + b2_ref[...], 0.0)
    f_ref[...] = f2
    preds_ref[...] = jnp.dot(f2, wf_ref[...], preferred_element_type=f32) + bf_ref[...]


def kernel(x, proxies, W_gat, att_src, att_dst, b_gat, W1, b1, W2, b2, Wf, bf):
    asrc_f = att_src.reshape(GOUT)
    adst_f = att_dst.reshape(GOUT)
    m0 = np.zeros((GOUT,), np.float32)
    m0[:C] = 1.0
    m0 = jnp.asarray(m0)
    m1 = 1.0 - m0

    M2 = jnp.zeros((GOUT, W), jnp.float32)
    M2 = M2.at[:, 0:8].set(jnp.broadcast_to((adst_f * m0)[:, None], (GOUT, 8)))
    M2 = M2.at[:, 8].set((asrc_f + adst_f) * m0)
    M2 = M2.at[:, 16:24].set(jnp.broadcast_to((adst_f * m1)[:, None], (GOUT, 8)))
    M2 = M2.at[:, 24].set((asrc_f + adst_f) * m1)

    as0 = (asrc_f * m0).reshape(1, GOUT)
    as1 = (asrc_f * m1).reshape(1, GOUT)

    grid = (N // BLOCK,)
    row_spec = pl.BlockSpec((BLOCK, EMBED), lambda i: (i, 0))

    def full(shape):
        return pl.BlockSpec(shape, lambda i: tuple(0 for _ in shape))

    preds, f = pl.pallas_call(
        _fused_kernel,
        grid=grid,
        in_specs=[
            row_spec,
            full((P, EMBED)),
            full((EMBED, GOUT)),
            full((GOUT, W)),
            full((1, GOUT)),
            full((1, GOUT)),
            full((P, W)),
            full((P, W)),
            full((1, W)),
            full((W, 8)),
            full((8, W)),
            full((W, P)),
            full((W, GOUT)),
            full((W, GOUT)),
            full((1, GOUT)),
            full((GOUT, HID)),
            full((1, HID)),
            full((HID, EMBED)),
            full((1, EMBED)),
            full((EMBED, OUTD)),
            full((1, OUTD)),
        ],
        out_specs=[pl.BlockSpec((BLOCK, OUTD), lambda i: (i, 0)),
                   pl.BlockSpec((BLOCK, EMBED), lambda i: (i, 0))],
        out_shape=[
            jax.ShapeDtypeStruct((N, OUTD), jnp.float32),
            jax.ShapeDtypeStruct((N, EMBED), jnp.float32),
        ],
        scratch_shapes=[
            pltpu.VMEM((EMBED, W), jnp.float32),
            pltpu.VMEM((W, 2 * GOUT), jnp.float32),
            pltpu.VMEM((1, W), jnp.float32),
        ],
    )(
        x,
        proxies,
        W_gat,
        M2,
        as0,
        as1,
        jnp.asarray(_G0_np),
        jnp.asarray(_G1_np),
        jnp.asarray(_PAD_np),
        jnp.asarray(_S_np),
        jnp.asarray(_ST_np),
        jnp.asarray(_E_np),
        jnp.asarray(_HPmask_np),
        jnp.asarray(_Ssel_np),
        b_gat.reshape(1, GOUT),
        W1,
        b1.reshape(1, HID),
        W2,
        b2.reshape(1, EMBED),
        Wf,
        bf.reshape(1, OUTD),
    )
    return preds, f


# speedup vs baseline: 587.1015x; 1.0478x over previous
import numpy as np
import jax
import jax.numpy as jnp
from jax.experimental import pallas as pl
from jax.experimental.pallas import tpu as pltpu

P = 8
N = 10000
EMBED = 128
H = 2
C = H * EMBED
GOUT = H * C
HID = 4 * EMBED
OUTD = 128

BLOCK = 2000
W = 32
NEG = -1e30

_S_np = np.zeros((W, 8), np.float32)
_S_np[0:9, 0] = 1.0
_S_np[16:25, 1] = 1.0
_ST_np = np.zeros((8, W), np.float32)
_ST_np[0, 0:9] = 1.0
_ST_np[1, 16:25] = 1.0
_E_np = np.zeros((W, P), np.float32)
_E_np[0:8, :] = np.eye(P, dtype=np.float32)
_E_np[16:24, :] = np.eye(P, dtype=np.float32)
_HPmask_np = np.zeros((W, GOUT), np.float32)
_HPmask_np[0:8, :C] = 1.0
_HPmask_np[16:24, C:] = 1.0
_Ssel_np = np.zeros((W, GOUT), np.float32)
_Ssel_np[8, :C] = 1.0
_Ssel_np[24, C:] = 1.0
_PAD_np = np.full((1, W), NEG, np.float32)
_PAD_np[0, 0:9] = 0.0
_PAD_np[0, 16:25] = 0.0
_G0_np = np.zeros((P, W), np.float32)
_G0_np[:, 0:8] = np.eye(P, dtype=np.float32)
_G1_np = np.zeros((P, W), np.float32)
_G1_np[:, 16:24] = np.eye(P, dtype=np.float32)


def _fused_kernel(x_ref, p_ref, wg_ref, m2_ref, as0_ref, as1_ref,
                  g0_ref, g1_ref, pad_ref, s_ref, st_ref, e_ref,
                  hpm_ref, ssel_ref, bg_ref,
                  w1_ref, b1_ref, w2_ref, b2_ref, wf_ref, bf_ref,
                  preds_ref, f_ref,
                  wgm2_scr, hpw2_scr, apvec_scr):
    f32 = jnp.float32

    @pl.when(pl.program_id(0) == 0)
    def _prep():
        wg = wg_ref[...]
        hp = jnp.dot(p_ref[...], wg, preferred_element_type=f32)
        dn = (((1,), (1,)), ((), ()))
        ap0 = jax.lax.dot_general(as0_ref[...], hp, dn, preferred_element_type=f32)
        ap1 = jax.lax.dot_general(as1_ref[...], hp, dn, preferred_element_type=f32)
        apvec_scr[...] = (jnp.dot(ap0, g0_ref[...], preferred_element_type=f32)
                          + jnp.dot(ap1, g1_ref[...], preferred_element_type=f32)
                          + pad_ref[...])
        wgm2_scr[...] = jnp.dot(wg, m2_ref[...], preferred_element_type=f32)
        hpw = jnp.dot(e_ref[...], hp, preferred_element_type=f32) * hpm_ref[...]
        hpw2_scr[...] = jnp.concatenate([hpw, ssel_ref[...]], axis=1)

    xb = x_ref[...]
    h = jnp.dot(xb, wg_ref[...], preferred_element_type=f32)

    z = jnp.dot(xb, wgm2_scr[...], preferred_element_type=f32) + apvec_scr[...]
    logits = jnp.maximum(z, 0.2 * z)
    m = jnp.max(logits, axis=1, keepdims=True)
    e = jnp.exp(logits - m)
    den = jnp.dot(e, s_ref[...], preferred_element_type=f32) + 1e-16
    rc = jnp.dot(1.0 / den, st_ref[...], preferred_element_type=f32)
    w = e * rc

    out = jnp.dot(w, hpw2_scr[...], preferred_element_type=f32)
    gat = out[:, :GOUT] + out[:, GOUT:] * h + bg_ref[...]

    f0 = jnp.maximum(gat, 0.0)
    f1 = jnp.maximum(jnp.dot(f0, w1_ref[...], preferred_element_type=f32) + b1_ref[...], 0.0)
    f2 = jnp.maximum(jnp.dot(f1, w2_ref[...], preferred_element_type=f32) + b2_ref[...], 0.0)
    f_ref[...] = f2
    preds_ref[...] = jnp.dot(f2, wf_ref[...], preferred_element_type=f32) + bf_ref[...]


def kernel(x, proxies, W_gat, att_src, att_dst, b_gat, W1, b1, W2, b2, Wf, bf):
    asrc_f = att_src.reshape(GOUT)
    adst_f = att_dst.reshape(GOUT)
    m0 = np.zeros((GOUT,), np.float32)
    m0[:C] = 1.0
    m0 = jnp.asarray(m0)
    m1 = 1.0 - m0

    M2 = jnp.zeros((GOUT, W), jnp.float32)
    M2 = M2.at[:, 0:8].set(jnp.broadcast_to((adst_f * m0)[:, None], (GOUT, 8)))
    M2 = M2.at[:, 8].set((asrc_f + adst_f) * m0)
    M2 = M2.at[:, 16:24].set(jnp.broadcast_to((adst_f * m1)[:, None], (GOUT, 8)))
    M2 = M2.at[:, 24].set((asrc_f + adst_f) * m1)

    as0 = (asrc_f * m0).reshape(1, GOUT)
    as1 = (asrc_f * m1).reshape(1, GOUT)

    grid = (N // BLOCK,)
    row_spec = pl.BlockSpec((BLOCK, EMBED), lambda i: (i, 0))

    def full(shape):
        return pl.BlockSpec(shape, lambda i: tuple(0 for _ in shape))

    preds, f = pl.pallas_call(
        _fused_kernel,
        grid=grid,
        in_specs=[
            row_spec,
            full((P, EMBED)),
            full((EMBED, GOUT)),
            full((GOUT, W)),
            full((1, GOUT)),
            full((1, GOUT)),
            full((P, W)),
            full((P, W)),
            full((1, W)),
            full((W, 8)),
            full((8, W)),
            full((W, P)),
            full((W, GOUT)),
            full((W, GOUT)),
            full((1, GOUT)),
            full((GOUT, HID)),
            full((1, HID)),
            full((HID, EMBED)),
            full((1, EMBED)),
            full((EMBED, OUTD)),
            full((1, OUTD)),
        ],
        out_specs=[pl.BlockSpec((BLOCK, OUTD), lambda i: (i, 0)),
                   pl.BlockSpec((BLOCK, EMBED), lambda i: (i, 0))],
        out_shape=[
            jax.ShapeDtypeStruct((N, OUTD), jnp.float32),
            jax.ShapeDtypeStruct((N, EMBED), jnp.float32),
        ],
        scratch_shapes=[
            pltpu.VMEM((EMBED, W), jnp.float32),
            pltpu.VMEM((W, 2 * GOUT), jnp.float32),
            pltpu.VMEM((1, W), jnp.float32),
        ],
    )(
        x,
        proxies,
        W_gat,
        M2,
        as0,
        as1,
        jnp.asarray(_G0_np),
        jnp.asarray(_G1_np),
        jnp.asarray(_PAD_np),
        jnp.asarray(_S_np),
        jnp.asarray(_ST_np),
        jnp.asarray(_E_np),
        jnp.asarray(_HPmask_np),
        jnp.asarray(_Ssel_np),
        b_gat.reshape(1, GOUT),
        W1,
        b1.reshape(1, HID),
        W2,
        b2.reshape(1, EMBED),
        Wf,
        bf.reshape(1, OUTD),
    )
    return preds, f


# speedup vs baseline: 597.9871x; 1.0185x over previous
import numpy as np
import jax
import jax.numpy as jnp
from jax.experimental import pallas as pl
from jax.experimental.pallas import tpu as pltpu

P = 8
N = 10000
EMBED = 128
H = 2
C = H * EMBED
GOUT = H * C
HID = 4 * EMBED
OUTD = 128

BLOCK = 5000
W = 32
NEG = -1e30

_S_np = np.zeros((W, 8), np.float32)
_S_np[0:9, 0] = 1.0
_S_np[16:25, 1] = 1.0
_ST_np = np.zeros((8, W), np.float32)
_ST_np[0, 0:9] = 1.0
_ST_np[1, 16:25] = 1.0
_E_np = np.zeros((W, P), np.float32)
_E_np[0:8, :] = np.eye(P, dtype=np.float32)
_E_np[16:24, :] = np.eye(P, dtype=np.float32)
_HPmask_np = np.zeros((W, GOUT), np.float32)
_HPmask_np[0:8, :C] = 1.0
_HPmask_np[16:24, C:] = 1.0
_Ssel_np = np.zeros((W, GOUT), np.float32)
_Ssel_np[8, :C] = 1.0
_Ssel_np[24, C:] = 1.0
_PAD_np = np.full((1, W), NEG, np.float32)
_PAD_np[0, 0:9] = 0.0
_PAD_np[0, 16:25] = 0.0
_G0_np = np.zeros((P, W), np.float32)
_G0_np[:, 0:8] = np.eye(P, dtype=np.float32)
_G1_np = np.zeros((P, W), np.float32)
_G1_np[:, 16:24] = np.eye(P, dtype=np.float32)


def _fused_kernel(x_ref, p_ref, wg_ref, m2_ref, as0_ref, as1_ref,
                  g0_ref, g1_ref, pad_ref, s_ref, st_ref, e_ref,
                  hpm_ref, ssel_ref, bg_ref,
                  w1_ref, b1_ref, w2_ref, b2_ref, wf_ref, bf_ref,
                  preds_ref, f_ref,
                  wgm2_scr, hpw2_scr, apvec_scr):
    f32 = jnp.float32

    @pl.when(pl.program_id(0) == 0)
    def _prep():
        wg = wg_ref[...]
        hp = jnp.dot(p_ref[...], wg, preferred_element_type=f32)
        dn = (((1,), (1,)), ((), ()))
        ap0 = jax.lax.dot_general(as0_ref[...], hp, dn, preferred_element_type=f32)
        ap1 = jax.lax.dot_general(as1_ref[...], hp, dn, preferred_element_type=f32)
        apvec_scr[...] = (jnp.dot(ap0, g0_ref[...], preferred_element_type=f32)
                          + jnp.dot(ap1, g1_ref[...], preferred_element_type=f32)
                          + pad_ref[...])
        wgm2_scr[...] = jnp.dot(wg, m2_ref[...], preferred_element_type=f32)
        hpw = jnp.dot(e_ref[...], hp, preferred_element_type=f32) * hpm_ref[...]
        hpw2_scr[...] = jnp.concatenate([hpw, ssel_ref[...]], axis=1)

    xb = x_ref[...]
    h = jnp.dot(xb, wg_ref[...], preferred_element_type=f32)

    z = jnp.dot(xb, wgm2_scr[...], preferred_element_type=f32) + apvec_scr[...]
    logits = jnp.maximum(z, 0.2 * z)
    m = jnp.max(logits, axis=1, keepdims=True)
    e = jnp.exp(logits - m)
    den = jnp.dot(e, s_ref[...], preferred_element_type=f32) + 1e-16
    rc = jnp.dot(1.0 / den, st_ref[...], preferred_element_type=f32)
    w = e * rc

    out = jnp.dot(w, hpw2_scr[...], preferred_element_type=f32)
    gat = out[:, :GOUT] + out[:, GOUT:] * h + bg_ref[...]

    f0 = jnp.maximum(gat, 0.0)
    f1 = jnp.maximum(jnp.dot(f0, w1_ref[...], preferred_element_type=f32) + b1_ref[...], 0.0)
    f2 = jnp.maximum(jnp.dot(f1, w2_ref[...], preferred_element_type=f32) + b2_ref[...], 0.0)
    f_ref[...] = f2
    preds_ref[...] = jnp.dot(f2, wf_ref[...], preferred_element_type=f32) + bf_ref[...]


def kernel(x, proxies, W_gat, att_src, att_dst, b_gat, W1, b1, W2, b2, Wf, bf):
    asrc_f = att_src.reshape(GOUT)
    adst_f = att_dst.reshape(GOUT)
    m0 = np.zeros((GOUT,), np.float32)
    m0[:C] = 1.0
    m0 = jnp.asarray(m0)
    m1 = 1.0 - m0

    M2 = jnp.zeros((GOUT, W), jnp.float32)
    M2 = M2.at[:, 0:8].set(jnp.broadcast_to((adst_f * m0)[:, None], (GOUT, 8)))
    M2 = M2.at[:, 8].set((asrc_f + adst_f) * m0)
    M2 = M2.at[:, 16:24].set(jnp.broadcast_to((adst_f * m1)[:, None], (GOUT, 8)))
    M2 = M2.at[:, 24].set((asrc_f + adst_f) * m1)

    as0 = (asrc_f * m0).reshape(1, GOUT)
    as1 = (asrc_f * m1).reshape(1, GOUT)

    grid = (N // BLOCK,)
    row_spec = pl.BlockSpec((BLOCK, EMBED), lambda i: (i, 0))

    def full(shape):
        return pl.BlockSpec(shape, lambda i: tuple(0 for _ in shape))

    preds, f = pl.pallas_call(
        _fused_kernel,
        grid=grid,
        in_specs=[
            row_spec,
            full((P, EMBED)),
            full((EMBED, GOUT)),
            full((GOUT, W)),
            full((1, GOUT)),
            full((1, GOUT)),
            full((P, W)),
            full((P, W)),
            full((1, W)),
            full((W, 8)),
            full((8, W)),
            full((W, P)),
            full((W, GOUT)),
            full((W, GOUT)),
            full((1, GOUT)),
            full((GOUT, HID)),
            full((1, HID)),
            full((HID, EMBED)),
            full((1, EMBED)),
            full((EMBED, OUTD)),
            full((1, OUTD)),
        ],
        out_specs=[pl.BlockSpec((BLOCK, OUTD), lambda i: (i, 0)),
                   pl.BlockSpec((BLOCK, EMBED), lambda i: (i, 0))],
        out_shape=[
            jax.ShapeDtypeStruct((N, OUTD), jnp.float32),
            jax.ShapeDtypeStruct((N, EMBED), jnp.float32),
        ],
        scratch_shapes=[
            pltpu.VMEM((EMBED, W), jnp.float32),
            pltpu.VMEM((W, 2 * GOUT), jnp.float32),
            pltpu.VMEM((1, W), jnp.float32),
        ],
    )(
        x,
        proxies,
        W_gat,
        M2,
        as0,
        as1,
        jnp.asarray(_G0_np),
        jnp.asarray(_G1_np),
        jnp.asarray(_PAD_np),
        jnp.asarray(_S_np),
        jnp.asarray(_ST_np),
        jnp.asarray(_E_np),
        jnp.asarray(_HPmask_np),
        jnp.asarray(_Ssel_np),
        b_gat.reshape(1, GOUT),
        W1,
        b1.reshape(1, HID),
        W2,
        b2.reshape(1, EMBED),
        Wf,
        bf.reshape(1, OUTD),
    )
    return preds, f


# speedup vs baseline: 785.5461x; 1.3137x over previous
import numpy as np
import jax
import jax.numpy as jnp
from jax.experimental import pallas as pl
from jax.experimental.pallas import tpu as pltpu

P = 8
N = 10000
EMBED = 128
H = 2
C = H * EMBED
GOUT = H * C
HID = 4 * EMBED
OUTD = 128

BLOCK = 5000
W = 32
NEG = -1e30

_S_np = np.zeros((W, 8), np.float32)
_S_np[0:9, 0] = 1.0
_S_np[16:25, 1] = 1.0
_ST_np = np.zeros((8, W), np.float32)
_ST_np[0, 0:9] = 1.0
_ST_np[1, 16:25] = 1.0
_E_np = np.zeros((W, P), np.float32)
_E_np[0:8, :] = np.eye(P, dtype=np.float32)
_E_np[16:24, :] = np.eye(P, dtype=np.float32)
_HPmask_np = np.zeros((W, GOUT), np.float32)
_HPmask_np[0:8, :C] = 1.0
_HPmask_np[16:24, C:] = 1.0
_Ssel_np = np.zeros((W, GOUT), np.float32)
_Ssel_np[8, :C] = 1.0
_Ssel_np[24, C:] = 1.0
_PAD_np = np.full((1, W), NEG, np.float32)
_PAD_np[0, 0:9] = 0.0
_PAD_np[0, 16:25] = 0.0
_G0_np = np.zeros((P, W), np.float32)
_G0_np[:, 0:8] = np.eye(P, dtype=np.float32)
_G1_np = np.zeros((P, W), np.float32)
_G1_np[:, 16:24] = np.eye(P, dtype=np.float32)
_M0_np = np.zeros((1, GOUT), np.float32)
_M0_np[0, :C] = 1.0
_R4_np = np.zeros((4, W), np.float32)
_R4_np[0, 0:8] = 1.0
_R4_np[1, 8] = 1.0
_R4_np[2, 16:24] = 1.0
_R4_np[3, 24] = 1.0


def _fused_kernel(x_ref, p_ref, wg_ref, asrc_ref, adst_ref, m0_ref, r4_ref,
                  g0_ref, g1_ref, pad_ref, s_ref, st_ref, e_ref,
                  hpm_ref, ssel_ref, bg_ref,
                  w1_ref, b1_ref, w2_ref, b2_ref, wf_ref, bf_ref,
                  preds_ref, f_ref,
                  wgm2_scr, hpw2_scr, apvec_scr):
    f32 = jnp.float32
    dn = (((1,), (1,)), ((), ()))

    @pl.when(pl.program_id(0) == 0)
    def _prep():
        wg = wg_ref[...]
        asrc = asrc_ref[...]
        adst = adst_ref[...]
        m0 = m0_ref[...]
        m1 = 1.0 - m0
        cols = jnp.concatenate([adst * m0, (asrc + adst) * m0,
                                adst * m1, (asrc + adst) * m1], axis=0)
        w4 = jax.lax.dot_general(wg, cols, dn, preferred_element_type=f32)
        wgm2_scr[...] = jnp.dot(w4, r4_ref[...], preferred_element_type=f32)

        hp = jnp.dot(p_ref[...], wg, preferred_element_type=f32)
        ap0 = jax.lax.dot_general(asrc * m0, hp, dn, preferred_element_type=f32)
        ap1 = jax.lax.dot_general(asrc * m1, hp, dn, preferred_element_type=f32)
        apvec_scr[...] = (jnp.dot(ap0, g0_ref[...], preferred_element_type=f32)
                          + jnp.dot(ap1, g1_ref[...], preferred_element_type=f32)
                          + pad_ref[...])
        hpw = jnp.dot(e_ref[...], hp, preferred_element_type=f32) * hpm_ref[...]
        hpw2_scr[...] = jnp.concatenate([hpw, ssel_ref[...]], axis=1)

    xb = x_ref[...]
    h = jnp.dot(xb, wg_ref[...], preferred_element_type=f32)

    z = jnp.dot(xb, wgm2_scr[...], preferred_element_type=f32) + apvec_scr[...]
    logits = jnp.maximum(z, 0.2 * z)
    m = jnp.max(logits, axis=1, keepdims=True)
    e = jnp.exp(logits - m)
    den = jnp.dot(e, s_ref[...], preferred_element_type=f32) + 1e-16
    rc = jnp.dot(1.0 / den, st_ref[...], preferred_element_type=f32)
    w = e * rc

    out = jnp.dot(w, hpw2_scr[...], preferred_element_type=f32)
    gat = out[:, :GOUT] + out[:, GOUT:] * h + bg_ref[...]

    f0 = jnp.maximum(gat, 0.0)
    f1 = jnp.maximum(jnp.dot(f0, w1_ref[...], preferred_element_type=f32) + b1_ref[...], 0.0)
    f2 = jnp.maximum(jnp.dot(f1, w2_ref[...], preferred_element_type=f32) + b2_ref[...], 0.0)
    f_ref[...] = f2
    preds_ref[...] = jnp.dot(f2, wf_ref[...], preferred_element_type=f32) + bf_ref[...]


def kernel(x, proxies, W_gat, att_src, att_dst, b_gat, W1, b1, W2, b2, Wf, bf):
    grid = (N // BLOCK,)
    row_spec = pl.BlockSpec((BLOCK, EMBED), lambda i: (i, 0))

    def full(shape):
        return pl.BlockSpec(shape, lambda i: tuple(0 for _ in shape))

    preds, f = pl.pallas_call(
        _fused_kernel,
        grid=grid,
        in_specs=[
            row_spec,
            full((P, EMBED)),
            full((EMBED, GOUT)),
            full((1, GOUT)),
            full((1, GOUT)),
            full((1, GOUT)),
            full((4, W)),
            full((P, W)),
            full((P, W)),
            full((1, W)),
            full((W, 8)),
            full((8, W)),
            full((W, P)),
            full((W, GOUT)),
            full((W, GOUT)),
            full((1, GOUT)),
            full((GOUT, HID)),
            full((1, HID)),
            full((HID, EMBED)),
            full((1, EMBED)),
            full((EMBED, OUTD)),
            full((1, OUTD)),
        ],
        out_specs=[pl.BlockSpec((BLOCK, OUTD), lambda i: (i, 0)),
                   pl.BlockSpec((BLOCK, EMBED), lambda i: (i, 0))],
        out_shape=[
            jax.ShapeDtypeStruct((N, OUTD), jnp.float32),
            jax.ShapeDtypeStruct((N, EMBED), jnp.float32),
        ],
        scratch_shapes=[
            pltpu.VMEM((EMBED, W), jnp.float32),
            pltpu.VMEM((W, 2 * GOUT), jnp.float32),
            pltpu.VMEM((1, W), jnp.float32),
        ],
    )(
        x,
        proxies,
        W_gat,
        att_src.reshape(1, GOUT),
        att_dst.reshape(1, GOUT),
        jnp.asarray(_M0_np),
        jnp.asarray(_R4_np),
        jnp.asarray(_G0_np),
        jnp.asarray(_G1_np),
        jnp.asarray(_PAD_np),
        jnp.asarray(_S_np),
        jnp.asarray(_ST_np),
        jnp.asarray(_E_np),
        jnp.asarray(_HPmask_np),
        jnp.asarray(_Ssel_np),
        b_gat.reshape(1, GOUT),
        W1,
        b1.reshape(1, HID),
        W2,
        b2.reshape(1, EMBED),
        Wf,
        bf.reshape(1, OUTD),
    )
    return preds, f


# speedup vs baseline: 852.6503x; 1.0854x over previous
import numpy as np
import jax
import jax.numpy as jnp
from jax.experimental import pallas as pl
from jax.experimental.pallas import tpu as pltpu

P = 8
N = 10000
EMBED = 128
H = 2
C = H * EMBED
GOUT = H * C
HID = 4 * EMBED
OUTD = 128

BLOCK = 5000
W = 32
NEG = -1e30

_S_np = np.zeros((W, 8), np.float32)
_S_np[0:9, 0] = 1.0
_S_np[16:25, 1] = 1.0
_ST_np = np.zeros((8, W), np.float32)
_ST_np[0, 0:9] = 1.0
_ST_np[1, 16:25] = 1.0
_E_np = np.zeros((W, P), np.float32)
_E_np[0:8, :] = np.eye(P, dtype=np.float32)
_E_np[16:24, :] = np.eye(P, dtype=np.float32)
_HPmask_np = np.zeros((W, GOUT), np.float32)
_HPmask_np[0:8, :C] = 1.0
_HPmask_np[16:24, C:] = 1.0
_PAD_np = np.full((1, W), NEG, np.float32)
_PAD_np[0, 0:9] = 0.0
_PAD_np[0, 16:25] = 0.0
_G0_np = np.zeros((P, W), np.float32)
_G0_np[:, 0:8] = np.eye(P, dtype=np.float32)
_G1_np = np.zeros((P, W), np.float32)
_G1_np[:, 16:24] = np.eye(P, dtype=np.float32)
_M0_np = np.zeros((1, GOUT), np.float32)
_M0_np[0, :C] = 1.0
_R4_np = np.zeros((4, W), np.float32)
_R4_np[0, 0:8] = 1.0
_R4_np[1, 8] = 1.0
_R4_np[2, 16:24] = 1.0
_R4_np[3, 24] = 1.0


def _fused_kernel(x_ref, p_ref, wg_ref, asrc_ref, adst_ref, m0_ref, r4_ref,
                  g0_ref, g1_ref, pad_ref, s_ref, st_ref, e_ref,
                  hpm_ref, bg_ref,
                  w1_ref, b1_ref, w2_ref, b2_ref, wf_ref, bf_ref,
                  preds_ref, f_ref,
                  wgm2_scr, hpw2_scr, apvec_scr):
    f32 = jnp.float32
    dn = (((1,), (1,)), ((), ()))

    @pl.when(pl.program_id(0) == 0)
    def _prep():
        wg = wg_ref[...]
        asrc = asrc_ref[...]
        adst = adst_ref[...]
        m0 = m0_ref[...]
        m1 = 1.0 - m0
        cols = jnp.concatenate([adst * m0, (asrc + adst) * m0,
                                adst * m1, (asrc + adst) * m1], axis=0)
        w4 = jax.lax.dot_general(wg, cols, dn, preferred_element_type=f32)
        wgm2_scr[...] = jnp.dot(w4, r4_ref[...], preferred_element_type=f32)

        hp = jnp.dot(p_ref[...], wg, preferred_element_type=f32)
        ap0 = jax.lax.dot_general(asrc * m0, hp, dn, preferred_element_type=f32)
        ap1 = jax.lax.dot_general(asrc * m1, hp, dn, preferred_element_type=f32)
        apvec_scr[...] = (jnp.dot(ap0, g0_ref[...], preferred_element_type=f32)
                          + jnp.dot(ap1, g1_ref[...], preferred_element_type=f32)
                          + pad_ref[...])
        hpw2_scr[...] = jnp.dot(e_ref[...], hp, preferred_element_type=f32) * hpm_ref[...]

    xb = x_ref[...]
    h = jnp.dot(xb, wg_ref[...], preferred_element_type=f32)

    z = jnp.dot(xb, wgm2_scr[...], preferred_element_type=f32) + apvec_scr[...]
    logits = jnp.maximum(z, 0.2 * z)
    m = jnp.max(logits, axis=1, keepdims=True)
    e = jnp.exp(logits - m)
    den = jnp.dot(e, s_ref[...], preferred_element_type=f32) + 1e-16
    rc = jnp.dot(1.0 / den, st_ref[...], preferred_element_type=f32)
    w = e * rc

    agg = jnp.dot(w, hpw2_scr[...], preferred_element_type=f32)
    selfc = jnp.concatenate([w[:, 8:9] * h[:, :C], w[:, 24:25] * h[:, C:]], axis=1)
    gat = agg + selfc + bg_ref[...]

    f0 = jnp.maximum(gat, 0.0)
    f1 = jnp.maximum(jnp.dot(f0, w1_ref[...], preferred_element_type=f32) + b1_ref[...], 0.0)
    f2 = jnp.maximum(jnp.dot(f1, w2_ref[...], preferred_element_type=f32) + b2_ref[...], 0.0)
    f_ref[...] = f2
    preds_ref[...] = jnp.dot(f2, wf_ref[...], preferred_element_type=f32) + bf_ref[...]


def kernel(x, proxies, W_gat, att_src, att_dst, b_gat, W1, b1, W2, b2, Wf, bf):
    grid = (N // BLOCK,)
    row_spec = pl.BlockSpec((BLOCK, EMBED), lambda i: (i, 0))

    def full(shape):
        return pl.BlockSpec(shape, lambda i: tuple(0 for _ in shape))

    preds, f = pl.pallas_call(
        _fused_kernel,
        grid=grid,
        in_specs=[
            row_spec,
            full((P, EMBED)),
            full((EMBED, GOUT)),
            full((1, GOUT)),
            full((1, GOUT)),
            full((1, GOUT)),
            full((4, W)),
            full((P, W)),
            full((P, W)),
            full((1, W)),
            full((W, 8)),
            full((8, W)),
            full((W, P)),
            full((W, GOUT)),
            full((1, GOUT)),
            full((GOUT, HID)),
            full((1, HID)),
            full((HID, EMBED)),
            full((1, EMBED)),
            full((EMBED, OUTD)),
            full((1, OUTD)),
        ],
        out_specs=[pl.BlockSpec((BLOCK, OUTD), lambda i: (i, 0)),
                   pl.BlockSpec((BLOCK, EMBED), lambda i: (i, 0))],
        out_shape=[
            jax.ShapeDtypeStruct((N, OUTD), jnp.float32),
            jax.ShapeDtypeStruct((N, EMBED), jnp.float32),
        ],
        scratch_shapes=[
            pltpu.VMEM((EMBED, W), jnp.float32),
            pltpu.VMEM((W, GOUT), jnp.float32),
            pltpu.VMEM((1, W), jnp.float32),
        ],
    )(
        x,
        proxies,
        W_gat,
        att_src.reshape(1, GOUT),
        att_dst.reshape(1, GOUT),
        jnp.asarray(_M0_np),
        jnp.asarray(_R4_np),
        jnp.asarray(_G0_np),
        jnp.asarray(_G1_np),
        jnp.asarray(_PAD_np),
        jnp.asarray(_S_np),
        jnp.asarray(_ST_np),
        jnp.asarray(_E_np),
        jnp.asarray(_HPmask_np),
        b_gat.reshape(1, GOUT),
        W1,
        b1.reshape(1, HID),
        W2,
        b2.reshape(1, EMBED),
        Wf,
        bf.reshape(1, OUTD),
    )
    return preds, f
